# baseline (speedup 1.0000x reference)
import jax
import jax.numpy as jnp
from jax.experimental import pallas as pl
from jax.experimental.pallas import tpu as pltpu

TN = 512


def _score_body(tok_ref, W_ref, bs_ref, w_ref, out_ref):
    x = tok_ref[0]
    z = jnp.dot(x, W_ref[...], preferred_element_type=jnp.float32)
    z = z + bs_ref[...][None, :]
    g = jax.nn.gelu(z)
    logit = jnp.dot(g, w_ref[...], preferred_element_type=jnp.float32)
    s = 1.0 / (1.0 + jnp.exp(-logit))
    out_ref[0] = s


def kernel(tokens, W_sam, b_sam, w_score, b_score):
    B, N, D = tokens.shape
    S = W_sam.shape[1]
    k = max(1, min(int(0.482 * N), N))
    scores = pl.pallas_call(
        _score_body,
        grid=(B, N // TN),
        in_specs=[
            pl.BlockSpec((1, TN, D), lambda b, t: (b, t, 0)),
            pl.BlockSpec((D, S), lambda b, t: (0, 0)),
            pl.BlockSpec((S,), lambda b, t: (0,)),
            pl.BlockSpec((S, 1), lambda b, t: (0, 0)),
        ],
        out_specs=pl.BlockSpec((1, TN, 1), lambda b, t: (b, t, 0)),
        out_shape=jax.ShapeDtypeStruct((B, N, 1), jnp.float32),
    )(tokens, W_sam, b_sam, w_score.reshape(S, 1)).reshape(B, N)
    _, topk_indices = jax.lax.top_k(scores, k)
    retained = jnp.take_along_axis(tokens, topk_indices[:, :, None], axis=1)
    return (retained, topk_indices, scores)
